# baseline (speedup 1.0000x reference)
import functools

import jax
import jax.numpy as jnp
from jax import lax
from jax.experimental import pallas as pl
from jax.experimental.pallas import tpu as pltpu
from jax.experimental.pallas import tpu_sc as plsc

N = 1000
NPAD = 1024
E = 64000
NC = 2
NS = 16
L = 16
NW = NC * NS
EPW = E // NW
GROUPS = EPW // L
NEG = -3.0e38

_f32 = jnp.float32

_GATHER_DN = lax.GatherDimensionNumbers(
    offset_dims=(), collapsed_slice_dims=(0,), start_index_map=(0,))


def _take(x, idx):
    return lax.gather(x, idx[:, None], _GATHER_DN, (1,),
                      mode=lax.GatherScatterMode.PROMISE_IN_BOUNDS)


_mesh = plsc.VectorSubcoreMesh(core_axis_name="c", subcore_axis_name="s")


@functools.partial(
    pl.kernel,
    out_type=[jax.ShapeDtypeStruct((NW, NPAD), _f32)] * 5,
    mesh=_mesh,
    compiler_params=pltpu.CompilerParams(needs_layout_passes=False),
    scratch_types=[
        pltpu.VMEM((EPW,), jnp.int32),
        pltpu.VMEM((EPW,), jnp.int32),
        pltpu.VMEM((2 * N,), _f32),
        pltpu.VMEM((NPAD,), _f32),
        pltpu.VMEM((NPAD,), _f32),
        pltpu.VMEM((NPAD,), _f32),
        pltpu.VMEM((8 * NPAD,), _f32),
        pltpu.VMEM((8 * NPAD,), _f32),
        pltpu.SemaphoreType.DMA,
        pltpu.SemaphoreType.DMA,
        pltpu.SemaphoreType.DMA,
    ],
)
def _sc_aggregate(edge_hbm, x_hbm,
                  o_sum0, o_sum1, o_cnt, o_max0, o_max1,
                  src_v, dst_v, x_v,
                  sum0_v, sum1_v, cnt_v, max0_v, max1_v, sems, semd, semx):
    wid = lax.axis_index("s") * NC + lax.axis_index("c")
    base = wid * EPW
    cps = pltpu.make_async_copy(edge_hbm.at[pl.ds(base, EPW)], src_v, sems)
    cpd = pltpu.make_async_copy(edge_hbm.at[pl.ds(E + base, EPW)], dst_v, semd)
    cpx = pltpu.make_async_copy(x_hbm, x_v, semx)
    cps.start()
    cpd.start()
    cpx.start()

    zeros16 = jnp.zeros((L,), _f32)
    ones16 = jnp.ones((L,), _f32)
    neg16 = jnp.full((L,), NEG, _f32)

    @plsc.parallel_loop(0, NPAD // L, unroll=2)
    def init_sums(j):
        off = j * L
        sum0_v[pl.ds(off, L)] = zeros16
        sum1_v[pl.ds(off, L)] = zeros16
        cnt_v[pl.ds(off, L)] = zeros16

    @plsc.parallel_loop(0, 8 * NPAD // L, unroll=2)
    def init_max(j):
        off = j * L
        max0_v[pl.ds(off, L)] = neg16
        max1_v[pl.ds(off, L)] = neg16

    cps.wait()
    cpd.wait()
    cpx.wait()

    @plsc.parallel_loop(0, GROUPS, unroll=4)
    def group_body(g):
        off = g * L
        koff = jnp.bitwise_and(g, 7) * NPAD
        d = dst_v[pl.ds(off, L)]
        s = src_v[pl.ds(off, L)]
        s2 = s + s
        g0 = plsc.load_gather(x_v, [s2])
        g1 = plsc.load_gather(x_v, [s2 + 1])
        plsc.addupdate_scatter(sum0_v, [d], g0)
        plsc.addupdate_scatter(sum1_v, [d], g1)
        plsc.addupdate_scatter(cnt_v, [d], ones16)
        g0s, d0 = plsc.sort_key_val(g0, d)
        i0 = d0 + koff
        c0 = plsc.load_gather(max0_v, [i0])
        plsc.store_scatter(max0_v, [i0], jnp.maximum(c0, g0s))
        g1s, d1 = plsc.sort_key_val(g1, d)
        i1 = d1 + koff
        c1 = plsc.load_gather(max1_v, [i1])
        plsc.store_scatter(max1_v, [i1], jnp.maximum(c1, g1s))

    @plsc.parallel_loop(0, NPAD // L, unroll=2)
    def merge_max(j):
        off = j * L
        a0 = max0_v[pl.ds(off, L)]
        a1 = max1_v[pl.ds(off, L)]
        for r in range(1, 8):
            a0 = jnp.maximum(a0, max0_v[pl.ds(off + r * NPAD, L)])
            a1 = jnp.maximum(a1, max1_v[pl.ds(off + r * NPAD, L)])
        max0_v[pl.ds(off, L)] = a0
        max1_v[pl.ds(off, L)] = a1

    pltpu.sync_copy(sum0_v, o_sum0.at[wid])
    pltpu.sync_copy(sum1_v, o_sum1.at[wid])
    pltpu.sync_copy(cnt_v, o_cnt.at[wid])
    pltpu.sync_copy(max0_v.at[pl.ds(0, NPAD)], o_max0.at[wid])
    pltpu.sync_copy(max1_v.at[pl.ds(0, NPAD)], o_max1.at[wid])


def _tc_body(wl0_ref, bl0_ref, wl1_ref, bl1_ref,
             p_sum0, p_sum1, p_cnt, p_max0, p_max1,
             x_ref, wr0_ref, wr1_ref,
             w0_hbm, b0_ref, w1_hbm, b1_ref, w2_hbm, b2_ref,
             emb_ref, out_ref,
             w0_v, w1_v, w2_v, sem0, sem1, sem2):
    cp0 = pltpu.make_async_copy(w0_hbm, w0_v, sem0)
    cp1 = pltpu.make_async_copy(w1_hbm, w1_v, sem1)
    cp2 = pltpu.make_async_copy(w2_hbm, w2_v, sem2)
    cp0.start()
    cp1.start()
    cp2.start()
    dims = (((1,), (1,)), ((), ()))
    sum0 = jnp.sum(p_sum0[...], axis=0, keepdims=True)[:, :N]
    sum1 = jnp.sum(p_sum1[...], axis=0, keepdims=True)[:, :N]
    cnt = jnp.sum(p_cnt[...], axis=0, keepdims=True)[:, :N]
    max0 = jnp.max(p_max0[...], axis=0, keepdims=True)[:, :N]
    max1 = jnp.max(p_max1[...], axis=0, keepdims=True)[:, :N]
    denom = jnp.maximum(cnt, 1.0)
    mean0 = sum0 / denom
    mean1 = sum1 / denom
    has = cnt > 0.0
    mx0 = jnp.where(has, max0, 0.0)
    mx1 = jnp.where(has, max1, 0.0)
    x = x_ref[...]
    xr0 = lax.dot_general(wr0_ref[...], x, dims, preferred_element_type=_f32)
    xr1 = lax.dot_general(wr1_ref[...], x, dims, preferred_element_type=_f32)
    xm = jnp.maximum(
        mean0 * wl0_ref[0, 0] + mean1 * wl0_ref[0, 1] + bl0_ref[0] + xr0, 0.0)
    xx = jnp.maximum(
        mx0 * wl1_ref[0, 0] + mx1 * wl1_ref[0, 1] + bl1_ref[0] + xr1, 0.0)
    emb = xm + xx
    emb_ref[...] = emb
    cp0.wait()
    h = jnp.maximum(
        lax.dot_general(emb, w0_v[...], dims, preferred_element_type=_f32)
        + b0_ref[...], 0.0)
    cp1.wait()
    h = jnp.maximum(
        lax.dot_general(h, w1_v[...], dims, preferred_element_type=_f32)
        + b1_ref[...], 0.0)
    cp2.wait()
    o = jnp.maximum(
        lax.dot_general(h, w2_v[...], dims, preferred_element_type=_f32)
        + b2_ref[...], 0.0)
    out_ref[...] = o


_smem = pl.BlockSpec(memory_space=pltpu.SMEM)
_any = pl.BlockSpec(memory_space=pl.ANY)
_mlp_call = pl.pallas_call(
    _tc_body,
    out_shape=[
        jax.ShapeDtypeStruct((1, N), _f32),
        jax.ShapeDtypeStruct((1, N - 1), _f32),
    ],
    in_specs=([_smem] * 4 + [pl.BlockSpec()] * 8
              + [_any, pl.BlockSpec(), _any, pl.BlockSpec(), _any,
                 pl.BlockSpec()]),
    scratch_shapes=[
        pltpu.VMEM((N, N), _f32),
        pltpu.VMEM((N, N), _f32),
        pltpu.VMEM((N - 1, N), _f32),
        pltpu.SemaphoreType.DMA,
        pltpu.SemaphoreType.DMA,
        pltpu.SemaphoreType.DMA,
    ],
)


def kernel(X, edge_index, Wl0, bl0, Wr0, Wl1, bl1, Wr1, W0, b0, W1, b1, W2, b2):
    s0, s1, ct, m0, m1 = _sc_aggregate(edge_index.reshape(2 * E), X.reshape(2 * N))
    emb, out = _mlp_call(
        Wl0, bl0, Wl1, bl1, s0, s1, ct, m0, m1,
        X, Wr0, Wr1,
        W0, b0.reshape(1, N), W1, b1.reshape(1, N), W2, b2.reshape(1, N - 1))
    return emb[0], out[0]


# speedup vs baseline: 1.0100x; 1.0100x over previous
import functools

import jax
import jax.numpy as jnp
from jax import lax
from jax.experimental import pallas as pl
from jax.experimental.pallas import tpu as pltpu
from jax.experimental.pallas import tpu_sc as plsc

N = 1000
NPAD = 1024
E = 64000
NC = 2
NS = 16
L = 16
NW = NC * NS
EPW = E // NW
GROUPS = EPW // L
NEG = -3.0e38

_f32 = jnp.float32

_mesh = plsc.VectorSubcoreMesh(core_axis_name="c", subcore_axis_name="s")


@functools.partial(
    pl.kernel,
    out_type=[jax.ShapeDtypeStruct((NW, NPAD), _f32)] * 5,
    mesh=_mesh,
    compiler_params=pltpu.CompilerParams(needs_layout_passes=False),
    scratch_types=[
        pltpu.VMEM((EPW,), jnp.int32),
        pltpu.VMEM((EPW,), jnp.int32),
        pltpu.VMEM((2 * N,), _f32),
        pltpu.VMEM((NPAD,), _f32),
        pltpu.VMEM((NPAD,), _f32),
        pltpu.VMEM((NPAD,), _f32),
        pltpu.VMEM((4 * NPAD,), _f32),
        pltpu.VMEM((4 * NPAD,), _f32),
        pltpu.SemaphoreType.DMA,
        pltpu.SemaphoreType.DMA,
        pltpu.SemaphoreType.DMA,
    ],
)
def _sc_aggregate(edge_hbm, x_hbm,
                  o_sum0, o_sum1, o_cnt, o_max0, o_max1,
                  src_v, dst_v, x_v,
                  sum0_v, sum1_v, cnt_v, max0_v, max1_v, sems, semd, semx):
    wid = lax.axis_index("s") * NC + lax.axis_index("c")
    base = wid * EPW
    cps = pltpu.make_async_copy(edge_hbm.at[pl.ds(base, EPW)], src_v, sems)
    cpd = pltpu.make_async_copy(edge_hbm.at[pl.ds(E + base, EPW)], dst_v, semd)
    cpx = pltpu.make_async_copy(x_hbm, x_v, semx)
    cps.start()
    cpd.start()
    cpx.start()

    zeros16 = jnp.zeros((L,), _f32)
    ones16 = jnp.ones((L,), _f32)
    neg16 = jnp.full((L,), NEG, _f32)

    @plsc.parallel_loop(0, NPAD // L, unroll=2)
    def init_sums(j):
        off = j * L
        sum0_v[pl.ds(off, L)] = zeros16
        sum1_v[pl.ds(off, L)] = zeros16
        cnt_v[pl.ds(off, L)] = zeros16

    @plsc.parallel_loop(0, 4 * NPAD // L, unroll=2)
    def init_max(j):
        off = j * L
        max0_v[pl.ds(off, L)] = neg16
        max1_v[pl.ds(off, L)] = neg16

    cps.wait()
    cpd.wait()
    cpx.wait()

    @plsc.parallel_loop(0, GROUPS, unroll=2)
    def group_body(g):
        off = g * L
        koff = jnp.bitwise_and(g, 3) * NPAD
        d = dst_v[pl.ds(off, L)]
        s = src_v[pl.ds(off, L)]
        s2 = s + s
        g0 = plsc.load_gather(x_v, [s2])
        g1 = plsc.load_gather(x_v, [s2 + 1])
        plsc.addupdate_scatter(sum0_v, [d], g0)
        plsc.addupdate_scatter(sum1_v, [d], g1)
        plsc.addupdate_scatter(cnt_v, [d], ones16)
        g0s, d0 = plsc.sort_key_val(g0, d)
        i0 = d0 + koff
        c0 = plsc.load_gather(max0_v, [i0])
        plsc.store_scatter(max0_v, [i0], jnp.maximum(c0, g0s))
        g1s, d1 = plsc.sort_key_val(g1, d)
        i1 = d1 + koff
        c1 = plsc.load_gather(max1_v, [i1])
        plsc.store_scatter(max1_v, [i1], jnp.maximum(c1, g1s))

    @plsc.parallel_loop(0, NPAD // L, unroll=2)
    def merge_max(j):
        off = j * L
        a0 = max0_v[pl.ds(off, L)]
        a1 = max1_v[pl.ds(off, L)]
        for r in range(1, 4):
            a0 = jnp.maximum(a0, max0_v[pl.ds(off + r * NPAD, L)])
            a1 = jnp.maximum(a1, max1_v[pl.ds(off + r * NPAD, L)])
        max0_v[pl.ds(off, L)] = a0
        max1_v[pl.ds(off, L)] = a1

    pltpu.sync_copy(sum0_v, o_sum0.at[wid])
    pltpu.sync_copy(sum1_v, o_sum1.at[wid])
    pltpu.sync_copy(cnt_v, o_cnt.at[wid])
    pltpu.sync_copy(max0_v.at[pl.ds(0, NPAD)], o_max0.at[wid])
    pltpu.sync_copy(max1_v.at[pl.ds(0, NPAD)], o_max1.at[wid])


def _tc_body(wl0_ref, bl0_ref, wl1_ref, bl1_ref,
             p_sum0, p_sum1, p_cnt, p_max0, p_max1,
             x_ref, wr0_ref, wr1_ref,
             w0_hbm, b0_ref, w1_hbm, b1_ref, w2_hbm, b2_ref,
             emb_ref, out_ref,
             w0_v, w1_v, w2_v, sem0, sem1, sem2):
    cp0 = pltpu.make_async_copy(w0_hbm, w0_v, sem0)
    cp1 = pltpu.make_async_copy(w1_hbm, w1_v, sem1)
    cp2 = pltpu.make_async_copy(w2_hbm, w2_v, sem2)
    cp0.start()
    cp1.start()
    cp2.start()
    dims = (((1,), (1,)), ((), ()))
    sum0 = jnp.sum(p_sum0[...], axis=0, keepdims=True)[:, :N]
    sum1 = jnp.sum(p_sum1[...], axis=0, keepdims=True)[:, :N]
    cnt = jnp.sum(p_cnt[...], axis=0, keepdims=True)[:, :N]
    max0 = jnp.max(p_max0[...], axis=0, keepdims=True)[:, :N]
    max1 = jnp.max(p_max1[...], axis=0, keepdims=True)[:, :N]
    denom = jnp.maximum(cnt, 1.0)
    mean0 = sum0 / denom
    mean1 = sum1 / denom
    has = cnt > 0.0
    mx0 = jnp.where(has, max0, 0.0)
    mx1 = jnp.where(has, max1, 0.0)
    x = x_ref[...]
    xr0 = lax.dot_general(wr0_ref[...], x, dims, preferred_element_type=_f32)
    xr1 = lax.dot_general(wr1_ref[...], x, dims, preferred_element_type=_f32)
    xm = jnp.maximum(
        mean0 * wl0_ref[0, 0] + mean1 * wl0_ref[0, 1] + bl0_ref[0] + xr0, 0.0)
    xx = jnp.maximum(
        mx0 * wl1_ref[0, 0] + mx1 * wl1_ref[0, 1] + bl1_ref[0] + xr1, 0.0)
    emb = xm + xx
    emb_ref[...] = emb
    cp0.wait()
    h = jnp.maximum(
        lax.dot_general(emb, w0_v[...], dims, preferred_element_type=_f32)
        + b0_ref[...], 0.0)
    cp1.wait()
    h = jnp.maximum(
        lax.dot_general(h, w1_v[...], dims, preferred_element_type=_f32)
        + b1_ref[...], 0.0)
    cp2.wait()
    o = jnp.maximum(
        lax.dot_general(h, w2_v[...], dims, preferred_element_type=_f32)
        + b2_ref[...], 0.0)
    out_ref[...] = o


_smem = pl.BlockSpec(memory_space=pltpu.SMEM)
_any = pl.BlockSpec(memory_space=pl.ANY)
_mlp_call = pl.pallas_call(
    _tc_body,
    out_shape=[
        jax.ShapeDtypeStruct((1, N), _f32),
        jax.ShapeDtypeStruct((1, N - 1), _f32),
    ],
    in_specs=([_smem] * 4 + [pl.BlockSpec()] * 8
              + [_any, pl.BlockSpec(), _any, pl.BlockSpec(), _any,
                 pl.BlockSpec()]),
    scratch_shapes=[
        pltpu.VMEM((N, N), _f32),
        pltpu.VMEM((N, N), _f32),
        pltpu.VMEM((N - 1, N), _f32),
        pltpu.SemaphoreType.DMA,
        pltpu.SemaphoreType.DMA,
        pltpu.SemaphoreType.DMA,
    ],
)


def kernel(X, edge_index, Wl0, bl0, Wr0, Wl1, bl1, Wr1, W0, b0, W1, b1, W2, b2):
    s0, s1, ct, m0, m1 = _sc_aggregate(edge_index.reshape(2 * E), X.reshape(2 * N))
    emb, out = _mlp_call(
        Wl0, bl0, Wl1, bl1, s0, s1, ct, m0, m1,
        X, Wr0, Wr1,
        W0, b0.reshape(1, N), W1, b1.reshape(1, N), W2, b2.reshape(1, N - 1))
    return emb[0], out[0]
